# Initial kernel scaffold; baseline (speedup 1.0000x reference)
import jax
import jax.numpy as jnp
from jax.experimental import pallas as pl


def kernel(node_record, t_record, emb_table, W1, b1, W2, b2):
    raise NotImplementedError("write your pallas kernel here")


# speedup vs baseline: 2.0015x; 2.0015x over previous
import functools

import jax
import jax.numpy as jnp
from jax import lax
from jax.experimental import pallas as pl
from jax.experimental.pallas import tpu as pltpu
from jax.experimental.pallas import tpu_sc as plsc

B, S = 4096, 200
VOCAB = 1000000
ENC_DIM = 16
N = B * S
ROW = 6

PACK = 8
PK = PACK * ROW
PH = PACK * 2 * ENC_DIM
PO = PACK * ENC_DIM
NPACKT = VOCAB // PACK
BRT = 5000
GRIDT = NPACKT // BRT

NC, NS = 2, 16
NW = NC * NS
PER_W = N // NW
CHUNK = 128
GROUP = 20
NCHUNK = PER_W // CHUNK
NGROUP = NCHUNK // GROUP


def _keys_body(node_ref, t_ref, out_ref):
    s = node_ref[...] + t_ref[...].astype(jnp.int32)
    out_ref[...] = jnp.where(s >= VOCAB, s - VOCAB, s)


_keys_call = pl.pallas_call(
    _keys_body,
    out_shape=jax.ShapeDtypeStruct((N // 128, 128), jnp.int32),
)


def _mlp_body(x_ref, w1_ref, b1_ref, w2_ref, b2_ref, o_ref):
    x = x_ref[...]
    h = jnp.maximum(
        jnp.dot(x, w1_ref[...], preferred_element_type=jnp.float32) + b1_ref[...], 0.0
    )
    o_ref[...] = (
        jnp.dot(h, w2_ref[...], preferred_element_type=jnp.float32) + b2_ref[...]
    )


_tab_call = pl.pallas_call(
    _mlp_body,
    grid=(GRIDT,),
    in_specs=[
        pl.BlockSpec((BRT, PK), lambda i: (i, 0)),
        pl.BlockSpec((PK, PH), lambda i: (0, 0)),
        pl.BlockSpec((1, PH), lambda i: (0, 0)),
        pl.BlockSpec((PH, PO), lambda i: (0, 0)),
        pl.BlockSpec((1, PO), lambda i: (0, 0)),
    ],
    out_specs=pl.BlockSpec((BRT, PO), lambda i: (i, 0)),
    out_shape=jax.ShapeDtypeStruct((NPACKT, PO), jnp.float32),
)


_sc_mesh = plsc.VectorSubcoreMesh(core_axis_name="c", subcore_axis_name="s")


@functools.partial(
    pl.kernel,
    out_type=jax.ShapeDtypeStruct((N, ENC_DIM), jnp.float32),
    mesh=_sc_mesh,
    scratch_types=[
        pltpu.VMEM((NCHUNK, CHUNK), jnp.int32),
        pltpu.VMEM((GROUP * CHUNK, ENC_DIM), jnp.float32),
        pltpu.SemaphoreType.DMA,
    ],
    compiler_params=pltpu.CompilerParams(use_tc_tiling_on_sc=False),
)
def _sc_gather(keys_hbm, ftab_hbm, out_hbm, idx_v, rows_v, sem):
    wid = lax.axis_index("s") * NC + lax.axis_index("c")
    pltpu.sync_copy(keys_hbm.at[pl.ds(wid * NCHUNK, NCHUNK)], idx_v)
    out_base = wid * PER_W

    def group_body(g, carry):
        cps = []
        for b in range(GROUP):
            cp = pltpu.async_copy(
                ftab_hbm.at[idx_v.at[g * GROUP + b]],
                rows_v.at[pl.ds(b * CHUNK, CHUNK)],
                sem,
            )
            cps.append(cp)
        for cp in cps:
            cp.wait()
        pltpu.sync_copy(
            rows_v,
            out_hbm.at[pl.ds(out_base + g * (GROUP * CHUNK), GROUP * CHUNK)],
        )
        return carry

    lax.fori_loop(0, NGROUP, group_body, 0)


def kernel(node_record, t_record, emb_table, W1, b1, W2, b2):
    node_f = node_record.astype(jnp.int32).reshape(N // 128, 128)
    t_f = t_record.reshape(N // 128, 128)
    keys = _keys_call(node_f, t_f)

    w1blk = jnp.zeros((ROW, 2 * ENC_DIM), jnp.float32)
    w1blk = w1blk.at[0:3, 0:ENC_DIM].set(W1).at[3:6, ENC_DIM:].set(W1)
    eye = jnp.eye(PACK, dtype=jnp.float32)
    w1big = jnp.kron(eye, w1blk)
    b1big = jnp.tile(jnp.concatenate([b1, b1]), PACK)[None, :]
    w2stack = jnp.concatenate([W2, W2], axis=0)
    w2big = jnp.kron(eye, w2stack)
    b2big = jnp.tile(2.0 * b2, PACK)[None, :]

    packed = emb_table.reshape(NPACKT, PK)
    ftab = _tab_call(packed, w1big, b1big, w2big, b2big)
    ftab = ftab.reshape(VOCAB, ENC_DIM)

    out = _sc_gather(keys, ftab)
    return out.reshape(B, S, ENC_DIM)
